# Initial kernel scaffold; baseline (speedup 1.0000x reference)
import jax
import jax.numpy as jnp
from jax.experimental import pallas as pl


def kernel(x, edge_index, W_gcn, b_gcn, ln_g, ln_b, Wl, Wr, att, b_gat):
    raise NotImplementedError("write your pallas kernel here")


# speedup vs baseline: 15.9984x; 15.9984x over previous
import functools

import jax
import jax.numpy as jnp
from jax import lax
from jax.experimental import pallas as pl
from jax.experimental.pallas import tpu as pltpu
from jax.experimental.pallas import tpu_sc as plsc

N = 10000
D = 128
NPAD = 10240
NC = 2
NS = 16
NW = NC * NS
EB = 128
IB = 8
CHUNK = NPAD // NS
GRID = 4
BLK = NPAD // GRID
F32 = jnp.float32


def _sc_mesh():
    return plsc.VectorSubcoreMesh(core_axis_name="c", subcore_axis_name="s")


def _deg_call(nb):
    @functools.partial(
        pl.kernel,
        out_type=jax.ShapeDtypeStruct((NC * NPAD,), F32),
        mesh=_sc_mesh(),
        scratch_types=[
            pltpu.VMEM((nb, EB), jnp.int32),
            pltpu.VMEM((EB,), F32),
            pltpu.VMEM((EB,), F32),
            pltpu.VMEM_SHARED((NPAD,), F32),
        ],
    )
    def deg_kernel(col_hbm, zeros1_hbm, out_hbm, colv, ones_v, zbuf, acc):
        c = lax.axis_index("c")
        s = lax.axis_index("s")
        w = c * NS + s
        pltpu.sync_copy(zeros1_hbm, zbuf)
        for i in range(CHUNK // EB):
            pltpu.sync_copy(zbuf, acc.at[pl.ds(s * CHUNK + i * EB, EB)])
        pltpu.sync_copy(col_hbm.at[w], colv)
        for k in range(EB // 16):
            ones_v[pl.ds(k * 16, 16)] = jnp.full((16,), 1.0, F32)
        plsc.subcore_barrier()

        def body(j, carry):
            pltpu.sync_copy(ones_v, acc.at[colv.at[j]], add=True)
            return carry

        lax.fori_loop(0, nb, body, 0)
        plsc.subcore_barrier()
        for i in range(CHUNK // EB):
            pltpu.sync_copy(acc.at[pl.ds(s * CHUNK + i * EB, EB)], zbuf)
            pltpu.sync_copy(
                zbuf, out_hbm.at[pl.ds(c * NPAD + s * CHUNK + i * EB, EB)])

    return deg_kernel


def _gcn_agg_call(nb):
    @functools.partial(
        pl.kernel,
        out_type=jax.ShapeDtypeStruct((NC, NPAD, D), F32),
        mesh=_sc_mesh(),
        scratch_types=[
            pltpu.VMEM((IB, EB), jnp.int32),
            pltpu.VMEM((IB, EB), jnp.int32),
            pltpu.VMEM((EB, D), F32),
            pltpu.VMEM((EB, D), F32),
            pltpu.SemaphoreType.DMA,
            pltpu.SemaphoreType.DMA,
            pltpu.VMEM_SHARED((NPAD, D), F32),
        ],
    )
    def gcn_kernel(y_hbm, row_hbm, col_hbm, zeros2_hbm, out_hbm,
                   rowb, colb, win0, win1, sem0, sem1, acc):
        c = lax.axis_index("c")
        s = lax.axis_index("s")
        w = c * NS + s
        pltpu.sync_copy(zeros2_hbm, win0)
        for i in range(CHUNK // EB):
            pltpu.sync_copy(win0, acc.at[pl.ds(s * CHUNK + i * EB, EB)])
        plsc.subcore_barrier()

        def refill(r, carry):
            pltpu.sync_copy(row_hbm.at[w, pl.ds(r * IB, IB)], rowb)
            pltpu.sync_copy(col_hbm.at[w, pl.ds(r * IB, IB)], colb)

            def body(bb, carry2):
                b0 = bb * 2
                b1 = b0 + 1
                cp0 = pltpu.async_copy(y_hbm.at[rowb.at[b0]], win0, sem0)
                cp1 = pltpu.async_copy(y_hbm.at[rowb.at[b1]], win1, sem1)
                cp0.wait()
                pltpu.sync_copy(win0, acc.at[colb.at[b0]], add=True)
                cp1.wait()
                pltpu.sync_copy(win1, acc.at[colb.at[b1]], add=True)
                return carry2

            lax.fori_loop(0, IB // 2, body, 0)
            return carry

        lax.fori_loop(0, nb // IB, refill, 0)
        plsc.subcore_barrier()
        for i in range(CHUNK // EB):
            pltpu.sync_copy(acc.at[pl.ds(s * CHUNK + i * EB, EB)], win0)
            pltpu.sync_copy(win0,
                            out_hbm.at[c, pl.ds(s * CHUNK + i * EB, EB)])

    return gcn_kernel


def _gat_edge_call(nb):
    @functools.partial(
        pl.kernel,
        out_type=[
            jax.ShapeDtypeStruct((NC, NPAD, D), F32),
            jax.ShapeDtypeStruct((NC * NPAD,), F32),
        ],
        mesh=_sc_mesh(),
        compiler_params=pltpu.CompilerParams(needs_layout_passes=False),
        scratch_types=[
            pltpu.VMEM((IB, EB), jnp.int32),
            pltpu.VMEM((IB, EB), jnp.int32),
            pltpu.VMEM((EB, D), F32),
            pltpu.VMEM((EB, D), F32),
            pltpu.VMEM((D,), F32),
            pltpu.VMEM((EB,), F32),
            pltpu.VMEM((EB,), F32),
            pltpu.VMEM((EB,), F32),
            pltpu.SemaphoreType.DMA,
            pltpu.SemaphoreType.DMA,
            pltpu.SemaphoreType.DMA,
            pltpu.VMEM_SHARED((NPAD, D), F32),
            pltpu.VMEM_SHARED((NPAD,), F32),
        ],
    )
    def gat_kernel(xl_hbm, xr_hbm, sl_hbm, att_hbm, row_hbm, col_hbm,
                   zeros2_hbm, zeros1_hbm, num_hbm, den_hbm,
                   rowb, colb, uw, xlw, attv, slw, scob, pv,
                   sem0, sem1, sem2, num, den):
        c = lax.axis_index("c")
        s = lax.axis_index("s")
        w = c * NS + s
        pltpu.sync_copy(zeros2_hbm, uw)
        pltpu.sync_copy(zeros1_hbm, pv)
        for i in range(CHUNK // EB):
            pltpu.sync_copy(uw, num.at[pl.ds(s * CHUNK + i * EB, EB)])
            pltpu.sync_copy(pv, den.at[pl.ds(s * CHUNK + i * EB, EB)])
        pltpu.sync_copy(att_hbm, attv)
        plsc.subcore_barrier()

        att_chunks = [attv[pl.ds(k * 16, 16)] for k in range(D // 16)]
        iota = lax.iota(jnp.int32, 16)
        lane15 = iota == 15

        def refill(rr, carry):
            pltpu.sync_copy(row_hbm.at[w, pl.ds(rr * IB, IB)], rowb)
            pltpu.sync_copy(col_hbm.at[w, pl.ds(rr * IB, IB)], colb)

            def batch(b, carry2):
                cpl = pltpu.async_copy(xl_hbm.at[rowb.at[b]], xlw, sem0)
                cps = pltpu.async_copy(sl_hbm.at[colb.at[b]], slw, sem2)
                cpu0 = pltpu.async_copy(xr_hbm.at[colb.at[b]], uw, sem1)
                cpu0.wait()
                cpu1 = pltpu.async_copy(xl_hbm.at[rowb.at[b]], uw, sem1,
                                        add=True)
                cpu1.wait()
                cpl.wait()
                cps.wait()

                def srow(r, carry3):
                    acc = jnp.zeros((16,), F32)
                    for k in range(D // 16):
                        u = uw[r, pl.ds(k * 16, 16)]
                        lr = jnp.where(u >= 0.0, u, 0.2 * u)
                        acc = acc + lr * att_chunks[k]
                    cum = plsc.cumsum(acc)
                    plsc.store_scatter(scob, [jnp.full((16,), r, jnp.int32)],
                                       cum, mask=lane15)
                    return carry3

                lax.fori_loop(0, EB, srow, 0)

                for k in range(EB // 16):
                    sck = scob[pl.ds(k * 16, 16)]
                    slc = slw[pl.ds(k * 16, 16)]
                    pv[pl.ds(k * 16, 16)] = jnp.exp(sck - slc)

                def sgroup(g, carry3):
                    pk = pv[pl.ds(g * 16, 16)]
                    for l in range(16):
                        r = g * 16 + l
                        pr = jnp.full((16,), pk[l], F32)
                        for k in range(D // 16):
                            a = xlw[r, pl.ds(k * 16, 16)]
                            xlw[r, pl.ds(k * 16, 16)] = a * pr
                    return carry3

                lax.fori_loop(0, EB // 16, sgroup, 0)

                pltpu.sync_copy(xlw, num.at[colb.at[b]], add=True)
                pltpu.sync_copy(pv, den.at[colb.at[b]], add=True)
                return carry2

            lax.fori_loop(0, IB, batch, 0)
            return carry

        lax.fori_loop(0, nb // IB, refill, 0)
        plsc.subcore_barrier()
        for i in range(CHUNK // EB):
            pltpu.sync_copy(num.at[pl.ds(s * CHUNK + i * EB, EB)], uw)
            pltpu.sync_copy(uw,
                            num_hbm.at[c, pl.ds(s * CHUNK + i * EB, EB)])
            pltpu.sync_copy(den.at[pl.ds(s * CHUNK + i * EB, EB)], pv)
            pltpu.sync_copy(
                pv, den_hbm.at[pl.ds(c * NPAD + s * CHUNK + i * EB, EB)])

    return gat_kernel


def _tc_prep(xp, W, d0, d1):
    def body(x_ref, w_ref, d0_ref, d1_ref, y_ref, dis_ref):
        deg = d0_ref[...] + d1_ref[...] + 1.0
        dis = lax.rsqrt(deg)
        xw = jnp.dot(x_ref[...], w_ref[...], preferred_element_type=F32)
        y_ref[...] = xw * dis
        dis_ref[...] = dis

    return pl.pallas_call(
        body,
        grid=(GRID,),
        in_specs=[
            pl.BlockSpec((BLK, D), lambda i: (i, 0)),
            pl.BlockSpec((D, D), lambda i: (0, 0)),
            pl.BlockSpec((BLK, 1), lambda i: (i, 0)),
            pl.BlockSpec((BLK, 1), lambda i: (i, 0)),
        ],
        out_specs=[
            pl.BlockSpec((BLK, D), lambda i: (i, 0)),
            pl.BlockSpec((BLK, 1), lambda i: (i, 0)),
        ],
        out_shape=[
            jax.ShapeDtypeStruct((NPAD, D), F32),
            jax.ShapeDtypeStruct((NPAD, 1), F32),
        ],
    )(xp, W, d0, d1)


def _tc_mid(acc0, acc1, y, dis, b_gcn, ln_g, ln_b, Wl, Wr, att):
    def body(a0, a1, y_ref, dis_ref, bg, lg, lb, wl, wr, at,
             xl_ref, xr_ref, sl_ref):
        pre = (a0[...] + a1[...] + y_ref[...]) * dis_ref[...] + bg[...]
        hr = jnp.maximum(pre, 0.0)
        mu = jnp.mean(hr, axis=1, keepdims=True)
        xc = hr - mu
        var = jnp.mean(xc * xc, axis=1, keepdims=True)
        h = xc * lax.rsqrt(var + 1e-5) * lg[...] + lb[...]
        xl = jnp.dot(h, wl[...], preferred_element_type=F32)
        xr = jnp.dot(h, wr[...], preferred_element_type=F32)
        u = xl + xr
        lr = jnp.where(u >= 0.0, u, 0.2 * u)
        sl_ref[...] = jnp.sum(lr * at[...], axis=1, keepdims=True)
        xl_ref[...] = xl
        xr_ref[...] = xr

    return pl.pallas_call(
        body,
        grid=(GRID,),
        in_specs=[
            pl.BlockSpec((BLK, D), lambda i: (i, 0)),
            pl.BlockSpec((BLK, D), lambda i: (i, 0)),
            pl.BlockSpec((BLK, D), lambda i: (i, 0)),
            pl.BlockSpec((BLK, 1), lambda i: (i, 0)),
            pl.BlockSpec((1, D), lambda i: (0, 0)),
            pl.BlockSpec((1, D), lambda i: (0, 0)),
            pl.BlockSpec((1, D), lambda i: (0, 0)),
            pl.BlockSpec((D, D), lambda i: (0, 0)),
            pl.BlockSpec((D, D), lambda i: (0, 0)),
            pl.BlockSpec((1, D), lambda i: (0, 0)),
        ],
        out_specs=[
            pl.BlockSpec((BLK, D), lambda i: (i, 0)),
            pl.BlockSpec((BLK, D), lambda i: (i, 0)),
            pl.BlockSpec((BLK, 1), lambda i: (i, 0)),
        ],
        out_shape=[
            jax.ShapeDtypeStruct((NPAD, D), F32),
            jax.ShapeDtypeStruct((NPAD, D), F32),
            jax.ShapeDtypeStruct((NPAD, 1), F32),
        ],
    )(acc0, acc1, y, dis, b_gcn, ln_g, ln_b, Wl, Wr, att)


def _tc_out(num0, num1, xl, den0, den1, b_gat):
    def body(n0, n1, xl_ref, d0_ref, d1_ref, bg, out_ref):
        dent = d0_ref[...] + d1_ref[...] + 1.0 + 1e-16
        o = (n0[...] + n1[...] + xl_ref[...]) / dent + bg[...]
        out_ref[...] = jnp.maximum(o, 0.0)

    return pl.pallas_call(
        body,
        grid=(GRID,),
        in_specs=[
            pl.BlockSpec((BLK, D), lambda i: (i, 0)),
            pl.BlockSpec((BLK, D), lambda i: (i, 0)),
            pl.BlockSpec((BLK, D), lambda i: (i, 0)),
            pl.BlockSpec((BLK, 1), lambda i: (i, 0)),
            pl.BlockSpec((BLK, 1), lambda i: (i, 0)),
            pl.BlockSpec((1, D), lambda i: (0, 0)),
        ],
        out_specs=pl.BlockSpec((BLK, D), lambda i: (i, 0)),
        out_shape=jax.ShapeDtypeStruct((NPAD, D), F32),
    )(num0, num1, xl, den0, den1, b_gat)


def kernel(x, edge_index, W_gcn, b_gcn, ln_g, ln_b, Wl, Wr, att, b_gat):
    ei = edge_index.astype(jnp.int32)
    E = ei.shape[1]
    per = NW * EB * IB
    nb = IB * (-(-E // per))
    Ep = NW * nb * EB
    pad = Ep - E
    dmy = N + (jnp.arange(pad, dtype=jnp.int32) % (NPAD - N))
    row = jnp.concatenate([ei[0], dmy]).reshape(NW, nb, EB)
    col = jnp.concatenate([ei[1], dmy]).reshape(NW, nb, EB)
    zeros2 = jnp.zeros((EB, D), F32)
    zeros1 = jnp.zeros((EB,), F32)
    xp = jnp.zeros((NPAD, D), F32).at[:N].set(x)

    degp = _deg_call(nb)(col, zeros1)
    y, dis = _tc_prep(xp, W_gcn, degp[:NPAD][:, None], degp[NPAD:][:, None])
    accp = _gcn_agg_call(nb)(y, row, col, zeros2)
    xl, xr, sl = _tc_mid(accp[0], accp[1], y, dis,
                         b_gcn.reshape(1, D), ln_g.reshape(1, D),
                         ln_b.reshape(1, D), Wl, Wr, att.reshape(1, D))
    nump, denp = _gat_edge_call(nb)(xl, xr, sl.reshape(NPAD), att,
                                    row, col, zeros2, zeros1)
    out = _tc_out(nump[0], nump[1], xl,
                  denp[:NPAD][:, None], denp[NPAD:][:, None],
                  b_gat.reshape(1, D))
    return out[:N]


# speedup vs baseline: 18.8198x; 1.1763x over previous
import functools

import jax
import jax.numpy as jnp
from jax import lax
from jax.experimental import pallas as pl
from jax.experimental.pallas import tpu as pltpu
from jax.experimental.pallas import tpu_sc as plsc

N = 10000
D = 128
NPAD = 10240
NC = 2
NS = 16
NW = NC * NS
EB = 128
IB = 8
CHUNK = NPAD // NS
GRID = 4
BLK = NPAD // GRID
F32 = jnp.float32


def _sc_mesh():
    return plsc.VectorSubcoreMesh(core_axis_name="c", subcore_axis_name="s")


def _deg_call(nb):
    @functools.partial(
        pl.kernel,
        out_type=jax.ShapeDtypeStruct((NC * NPAD,), F32),
        mesh=_sc_mesh(),
        scratch_types=[
            pltpu.VMEM((nb, EB), jnp.int32),
            pltpu.VMEM((EB,), F32),
            pltpu.VMEM((EB,), F32),
            pltpu.VMEM_SHARED((NPAD,), F32),
        ],
    )
    def deg_kernel(col_hbm, zeros1_hbm, out_hbm, colv, ones_v, zbuf, acc):
        c = lax.axis_index("c")
        s = lax.axis_index("s")
        w = c * NS + s
        pltpu.sync_copy(zeros1_hbm, zbuf)
        for i in range(CHUNK // EB):
            pltpu.sync_copy(zbuf, acc.at[pl.ds(s * CHUNK + i * EB, EB)])
        pltpu.sync_copy(col_hbm.at[w], colv)
        for k in range(EB // 16):
            ones_v[pl.ds(k * 16, 16)] = jnp.full((16,), 1.0, F32)
        plsc.subcore_barrier()

        def body(j, carry):
            pltpu.sync_copy(ones_v, acc.at[colv.at[j]], add=True)
            return carry

        lax.fori_loop(0, nb, body, 0)
        plsc.subcore_barrier()
        for i in range(CHUNK // EB):
            pltpu.sync_copy(acc.at[pl.ds(s * CHUNK + i * EB, EB)], zbuf)
            pltpu.sync_copy(
                zbuf, out_hbm.at[pl.ds(c * NPAD + s * CHUNK + i * EB, EB)])

    return deg_kernel


def _gcn_agg_call(nb):
    @functools.partial(
        pl.kernel,
        out_type=jax.ShapeDtypeStruct((NC, NPAD, D), F32),
        mesh=_sc_mesh(),
        scratch_types=[
            pltpu.VMEM((IB, EB), jnp.int32),
            pltpu.VMEM((IB, EB), jnp.int32),
            pltpu.VMEM((EB, D), F32),
            pltpu.VMEM((EB, D), F32),
            pltpu.SemaphoreType.DMA,
            pltpu.SemaphoreType.DMA,
            pltpu.VMEM_SHARED((NPAD, D), F32),
        ],
    )
    def gcn_kernel(y_hbm, row_hbm, col_hbm, zeros2_hbm, out_hbm,
                   rowb, colb, win0, win1, sem0, sem1, acc):
        c = lax.axis_index("c")
        s = lax.axis_index("s")
        w = c * NS + s
        pltpu.sync_copy(zeros2_hbm, win0)
        for i in range(CHUNK // EB):
            pltpu.sync_copy(win0, acc.at[pl.ds(s * CHUNK + i * EB, EB)])
        plsc.subcore_barrier()

        def refill(r, carry):
            pltpu.sync_copy(row_hbm.at[w, pl.ds(r * IB, IB)], rowb)
            pltpu.sync_copy(col_hbm.at[w, pl.ds(r * IB, IB)], colb)

            def body(bb, carry2):
                b0 = bb * 2
                b1 = b0 + 1
                cp0 = pltpu.async_copy(y_hbm.at[rowb.at[b0]], win0, sem0)
                cp1 = pltpu.async_copy(y_hbm.at[rowb.at[b1]], win1, sem1)
                cp0.wait()
                pltpu.sync_copy(win0, acc.at[colb.at[b0]], add=True)
                cp1.wait()
                pltpu.sync_copy(win1, acc.at[colb.at[b1]], add=True)
                return carry2

            lax.fori_loop(0, IB // 2, body, 0)
            return carry

        lax.fori_loop(0, nb // IB, refill, 0)
        plsc.subcore_barrier()
        for i in range(CHUNK // EB):
            pltpu.sync_copy(acc.at[pl.ds(s * CHUNK + i * EB, EB)], win0)
            pltpu.sync_copy(win0,
                            out_hbm.at[c, pl.ds(s * CHUNK + i * EB, EB)])

    return gcn_kernel


def _gat_edge_call(nb):
    @functools.partial(
        pl.kernel,
        out_type=[
            jax.ShapeDtypeStruct((NC, NPAD, D), F32),
            jax.ShapeDtypeStruct((NC * NPAD,), F32),
        ],
        mesh=_sc_mesh(),
        compiler_params=pltpu.CompilerParams(needs_layout_passes=False),
        scratch_types=[
            pltpu.VMEM((IB, EB), jnp.int32),
            pltpu.VMEM((IB, EB), jnp.int32),
            pltpu.VMEM((EB, D), F32),
            pltpu.VMEM((EB, D), F32),
            pltpu.VMEM((D,), F32),
            pltpu.VMEM((EB,), F32),
            pltpu.VMEM((EB,), F32),
            pltpu.VMEM((EB,), F32),
            pltpu.SemaphoreType.DMA,
            pltpu.SemaphoreType.DMA,
            pltpu.SemaphoreType.DMA,
            pltpu.VMEM_SHARED((NPAD, D), F32),
            pltpu.VMEM_SHARED((NPAD,), F32),
        ],
    )
    def gat_kernel(xl_hbm, xr_hbm, sl_hbm, att_hbm, row_hbm, col_hbm,
                   zeros2_hbm, zeros1_hbm, num_hbm, den_hbm,
                   rowb, colb, uw, xlw, attv, slw, scob, pv,
                   sem0, sem1, sem2, num, den):
        c = lax.axis_index("c")
        s = lax.axis_index("s")
        w = c * NS + s
        pltpu.sync_copy(zeros2_hbm, uw)
        pltpu.sync_copy(zeros1_hbm, pv)
        for i in range(CHUNK // EB):
            pltpu.sync_copy(uw, num.at[pl.ds(s * CHUNK + i * EB, EB)])
            pltpu.sync_copy(pv, den.at[pl.ds(s * CHUNK + i * EB, EB)])
        pltpu.sync_copy(att_hbm, attv)
        plsc.subcore_barrier()

        att_chunks = [attv[pl.ds(k * 16, 16)] for k in range(D // 16)]
        iota = lax.iota(jnp.int32, 16)
        lane15 = iota == 15

        def refill(rr, carry):
            pltpu.sync_copy(row_hbm.at[w, pl.ds(rr * IB, IB)], rowb)
            pltpu.sync_copy(col_hbm.at[w, pl.ds(rr * IB, IB)], colb)

            def batch(b, carry2):
                cpl = pltpu.async_copy(xl_hbm.at[rowb.at[b]], xlw, sem0)
                cps = pltpu.async_copy(sl_hbm.at[colb.at[b]], slw, sem2)
                cpu0 = pltpu.async_copy(xr_hbm.at[colb.at[b]], uw, sem1)
                cpu0.wait()
                cpu1 = pltpu.async_copy(xl_hbm.at[rowb.at[b]], uw, sem1,
                                        add=True)
                cpu1.wait()
                cpl.wait()
                cps.wait()

                @plsc.parallel_loop(0, EB, unroll=4)
                def srow(r):
                    acc = jnp.zeros((16,), F32)
                    for k in range(D // 16):
                        u = uw[r, pl.ds(k * 16, 16)]
                        lr = jnp.where(u >= 0.0, u, 0.2 * u)
                        acc = acc + lr * att_chunks[k]
                    cum = plsc.cumsum(acc)
                    plsc.store_scatter(scob, [jnp.full((16,), r, jnp.int32)],
                                       cum, mask=lane15)

                for k in range(EB // 16):
                    sck = scob[pl.ds(k * 16, 16)]
                    slc = slw[pl.ds(k * 16, 16)]
                    pv[pl.ds(k * 16, 16)] = jnp.exp(sck - slc)

                @plsc.parallel_loop(0, EB // 16, unroll=1)
                def sgroup(g):
                    pk = pv[pl.ds(g * 16, 16)]
                    for l in range(16):
                        r = g * 16 + l
                        pr = jnp.full((16,), pk[l], F32)
                        for k in range(D // 16):
                            a = xlw[r, pl.ds(k * 16, 16)]
                            xlw[r, pl.ds(k * 16, 16)] = a * pr

                pltpu.sync_copy(xlw, num.at[colb.at[b]], add=True)
                pltpu.sync_copy(pv, den.at[colb.at[b]], add=True)
                return carry2

            lax.fori_loop(0, IB, batch, 0)
            return carry

        lax.fori_loop(0, nb // IB, refill, 0)
        plsc.subcore_barrier()
        for i in range(CHUNK // EB):
            pltpu.sync_copy(num.at[pl.ds(s * CHUNK + i * EB, EB)], uw)
            pltpu.sync_copy(uw,
                            num_hbm.at[c, pl.ds(s * CHUNK + i * EB, EB)])
            pltpu.sync_copy(den.at[pl.ds(s * CHUNK + i * EB, EB)], pv)
            pltpu.sync_copy(
                pv, den_hbm.at[pl.ds(c * NPAD + s * CHUNK + i * EB, EB)])

    return gat_kernel


def _tc_prep(xp, W, d0, d1):
    def body(x_ref, w_ref, d0_ref, d1_ref, y_ref, dis_ref):
        deg = d0_ref[...] + d1_ref[...] + 1.0
        dis = lax.rsqrt(deg)
        xw = jnp.dot(x_ref[...], w_ref[...], preferred_element_type=F32)
        y_ref[...] = xw * dis
        dis_ref[...] = dis

    return pl.pallas_call(
        body,
        grid=(GRID,),
        in_specs=[
            pl.BlockSpec((BLK, D), lambda i: (i, 0)),
            pl.BlockSpec((D, D), lambda i: (0, 0)),
            pl.BlockSpec((BLK, 1), lambda i: (i, 0)),
            pl.BlockSpec((BLK, 1), lambda i: (i, 0)),
        ],
        out_specs=[
            pl.BlockSpec((BLK, D), lambda i: (i, 0)),
            pl.BlockSpec((BLK, 1), lambda i: (i, 0)),
        ],
        out_shape=[
            jax.ShapeDtypeStruct((NPAD, D), F32),
            jax.ShapeDtypeStruct((NPAD, 1), F32),
        ],
    )(xp, W, d0, d1)


def _tc_mid(acc0, acc1, y, dis, b_gcn, ln_g, ln_b, Wl, Wr, att):
    def body(a0, a1, y_ref, dis_ref, bg, lg, lb, wl, wr, at,
             xl_ref, xr_ref, sl_ref):
        pre = (a0[...] + a1[...] + y_ref[...]) * dis_ref[...] + bg[...]
        hr = jnp.maximum(pre, 0.0)
        mu = jnp.mean(hr, axis=1, keepdims=True)
        xc = hr - mu
        var = jnp.mean(xc * xc, axis=1, keepdims=True)
        h = xc * lax.rsqrt(var + 1e-5) * lg[...] + lb[...]
        xl = jnp.dot(h, wl[...], preferred_element_type=F32)
        xr = jnp.dot(h, wr[...], preferred_element_type=F32)
        u = xl + xr
        lr = jnp.where(u >= 0.0, u, 0.2 * u)
        sl_ref[...] = jnp.sum(lr * at[...], axis=1, keepdims=True)
        xl_ref[...] = xl
        xr_ref[...] = xr

    return pl.pallas_call(
        body,
        grid=(GRID,),
        in_specs=[
            pl.BlockSpec((BLK, D), lambda i: (i, 0)),
            pl.BlockSpec((BLK, D), lambda i: (i, 0)),
            pl.BlockSpec((BLK, D), lambda i: (i, 0)),
            pl.BlockSpec((BLK, 1), lambda i: (i, 0)),
            pl.BlockSpec((1, D), lambda i: (0, 0)),
            pl.BlockSpec((1, D), lambda i: (0, 0)),
            pl.BlockSpec((1, D), lambda i: (0, 0)),
            pl.BlockSpec((D, D), lambda i: (0, 0)),
            pl.BlockSpec((D, D), lambda i: (0, 0)),
            pl.BlockSpec((1, D), lambda i: (0, 0)),
        ],
        out_specs=[
            pl.BlockSpec((BLK, D), lambda i: (i, 0)),
            pl.BlockSpec((BLK, D), lambda i: (i, 0)),
            pl.BlockSpec((BLK, 1), lambda i: (i, 0)),
        ],
        out_shape=[
            jax.ShapeDtypeStruct((NPAD, D), F32),
            jax.ShapeDtypeStruct((NPAD, D), F32),
            jax.ShapeDtypeStruct((NPAD, 1), F32),
        ],
    )(acc0, acc1, y, dis, b_gcn, ln_g, ln_b, Wl, Wr, att)


def _tc_out(num0, num1, xl, den0, den1, b_gat):
    def body(n0, n1, xl_ref, d0_ref, d1_ref, bg, out_ref):
        dent = d0_ref[...] + d1_ref[...] + 1.0 + 1e-16
        o = (n0[...] + n1[...] + xl_ref[...]) / dent + bg[...]
        out_ref[...] = jnp.maximum(o, 0.0)

    return pl.pallas_call(
        body,
        grid=(GRID,),
        in_specs=[
            pl.BlockSpec((BLK, D), lambda i: (i, 0)),
            pl.BlockSpec((BLK, D), lambda i: (i, 0)),
            pl.BlockSpec((BLK, D), lambda i: (i, 0)),
            pl.BlockSpec((BLK, 1), lambda i: (i, 0)),
            pl.BlockSpec((BLK, 1), lambda i: (i, 0)),
            pl.BlockSpec((1, D), lambda i: (0, 0)),
        ],
        out_specs=pl.BlockSpec((BLK, D), lambda i: (i, 0)),
        out_shape=jax.ShapeDtypeStruct((NPAD, D), F32),
    )(num0, num1, xl, den0, den1, b_gat)


def kernel(x, edge_index, W_gcn, b_gcn, ln_g, ln_b, Wl, Wr, att, b_gat):
    ei = edge_index.astype(jnp.int32)
    E = ei.shape[1]
    per = NW * EB * IB
    nb = IB * (-(-E // per))
    Ep = NW * nb * EB
    pad = Ep - E
    dmy = N + (jnp.arange(pad, dtype=jnp.int32) % (NPAD - N))
    row = jnp.concatenate([ei[0], dmy]).reshape(NW, nb, EB)
    col = jnp.concatenate([ei[1], dmy]).reshape(NW, nb, EB)
    zeros2 = jnp.zeros((EB, D), F32)
    zeros1 = jnp.zeros((EB,), F32)
    xp = jnp.zeros((NPAD, D), F32).at[:N].set(x)

    degp = _deg_call(nb)(col, zeros1)
    y, dis = _tc_prep(xp, W_gcn, degp[:NPAD][:, None], degp[NPAD:][:, None])
    accp = _gcn_agg_call(nb)(y, row, col, zeros2)
    xl, xr, sl = _tc_mid(accp[0], accp[1], y, dis,
                         b_gcn.reshape(1, D), ln_g.reshape(1, D),
                         ln_b.reshape(1, D), Wl, Wr, att.reshape(1, D))
    nump, denp = _gat_edge_call(nb)(xl, xr, sl.reshape(NPAD), att,
                                    row, col, zeros2, zeros1)
    out = _tc_out(nump[0], nump[1], xl,
                  denp[:NPAD][:, None], denp[NPAD:][:, None],
                  b_gat.reshape(1, D))
    return out[:N]


# speedup vs baseline: 27.1242x; 1.4413x over previous
import functools

import jax
import jax.numpy as jnp
from jax import lax
from jax.experimental import pallas as pl
from jax.experimental.pallas import tpu as pltpu
from jax.experimental.pallas import tpu_sc as plsc

N = 10000
D = 128
NPAD = 10240
NC = 2
NS = 16
NW = NC * NS
EB = 128
IB = 8
EB5 = 64
IB5 = 32
CHUNK = NPAD // NS
GRID = 4
BLK = NPAD // GRID
F32 = jnp.float32


def _sc_mesh():
    return plsc.VectorSubcoreMesh(core_axis_name="c", subcore_axis_name="s")


_GD = lax.GatherDimensionNumbers(offset_dims=(), collapsed_slice_dims=(0,),
                                 start_index_map=(0,))


def _lane_pick(v, idx16):
    return lax.gather(v, idx16[:, None], _GD, (1,),
                      mode=lax.GatherScatterMode.PROMISE_IN_BOUNDS)


def _deg_call(nb):
    @functools.partial(
        pl.kernel,
        out_type=jax.ShapeDtypeStruct((NC * NPAD,), F32),
        mesh=_sc_mesh(),
        scratch_types=[
            pltpu.VMEM((nb, EB), jnp.int32),
            pltpu.VMEM((EB,), F32),
            pltpu.VMEM((EB,), F32),
            pltpu.VMEM_SHARED((NPAD,), F32),
        ],
    )
    def deg_kernel(col_hbm, zeros1_hbm, out_hbm, colv, ones_v, zbuf, acc):
        c = lax.axis_index("c")
        s = lax.axis_index("s")
        w = c * NS + s
        pltpu.sync_copy(zeros1_hbm, zbuf)
        for i in range(CHUNK // EB):
            pltpu.sync_copy(zbuf, acc.at[pl.ds(s * CHUNK + i * EB, EB)])
        pltpu.sync_copy(col_hbm.at[w], colv)
        for k in range(EB // 16):
            ones_v[pl.ds(k * 16, 16)] = jnp.full((16,), 1.0, F32)
        plsc.subcore_barrier()

        def body(j, carry):
            pltpu.sync_copy(ones_v, acc.at[colv.at[j]], add=True)
            return carry

        lax.fori_loop(0, nb, body, 0)
        plsc.subcore_barrier()
        for i in range(CHUNK // EB):
            pltpu.sync_copy(acc.at[pl.ds(s * CHUNK + i * EB, EB)], zbuf)
            pltpu.sync_copy(
                zbuf, out_hbm.at[pl.ds(c * NPAD + s * CHUNK + i * EB, EB)])

    return deg_kernel


def _gcn_agg_call(nb):
    @functools.partial(
        pl.kernel,
        out_type=jax.ShapeDtypeStruct((NC, NPAD, D), F32),
        mesh=_sc_mesh(),
        scratch_types=[
            pltpu.VMEM((IB, EB), jnp.int32),
            pltpu.VMEM((IB, EB), jnp.int32),
            pltpu.VMEM((EB, D), F32),
            pltpu.VMEM((EB, D), F32),
            pltpu.SemaphoreType.DMA,
            pltpu.SemaphoreType.DMA,
            pltpu.SemaphoreType.DMA,
            pltpu.SemaphoreType.DMA,
            pltpu.VMEM_SHARED((NPAD, D), F32),
        ],
    )
    def gcn_kernel(y_hbm, row_hbm, col_hbm, zeros2_hbm, out_hbm,
                   rowb, colb, win0, win1, semg0, semg1, sems0, sems1, acc):
        c = lax.axis_index("c")
        s = lax.axis_index("s")
        w = c * NS + s
        pltpu.sync_copy(zeros2_hbm, win0)
        for i in range(CHUNK // EB):
            pltpu.sync_copy(win0, acc.at[pl.ds(s * CHUNK + i * EB, EB)])
        plsc.subcore_barrier()

        def refill(r, carry):
            pltpu.sync_copy(row_hbm.at[w, pl.ds(r * IB, IB)], rowb)
            pltpu.sync_copy(col_hbm.at[w, pl.ds(r * IB, IB)], colb)
            pltpu.async_copy(y_hbm.at[rowb.at[0]], win0, semg0)

            def body(bb, carry2):
                b0 = bb * 2
                b1 = b0 + 1
                pltpu.make_async_copy(y_hbm.at[rowb.at[b0]],
                                      win0, semg0).wait()

                @pl.when(bb > 0)
                def _():
                    pltpu.make_async_copy(win1, acc.at[colb.at[0]],
                                          sems1).wait()

                pltpu.async_copy(y_hbm.at[rowb.at[b1]], win1, semg1)
                pltpu.async_copy(win0, acc.at[colb.at[b0]], sems0, add=True)
                pltpu.make_async_copy(y_hbm.at[rowb.at[b1]],
                                      win1, semg1).wait()
                pltpu.make_async_copy(win0, acc.at[colb.at[0]],
                                      sems0).wait()

                @pl.when(bb < IB // 2 - 1)
                def _():
                    pltpu.async_copy(y_hbm.at[rowb.at[b0 + 2]], win0, semg0)

                pltpu.async_copy(win1, acc.at[colb.at[b1]], sems1, add=True)
                return carry2

            lax.fori_loop(0, IB // 2, body, 0)
            pltpu.make_async_copy(win1, acc.at[colb.at[0]], sems1).wait()
            return carry

        lax.fori_loop(0, nb // IB, refill, 0)
        plsc.subcore_barrier()
        for i in range(CHUNK // EB):
            pltpu.sync_copy(acc.at[pl.ds(s * CHUNK + i * EB, EB)], win0)
            pltpu.sync_copy(win0,
                            out_hbm.at[c, pl.ds(s * CHUNK + i * EB, EB)])

    return gcn_kernel


def _gat_edge_call(nb5):
    @functools.partial(
        pl.kernel,
        out_type=[
            jax.ShapeDtypeStruct((NC, NPAD, D), F32),
            jax.ShapeDtypeStruct((NC * NPAD,), F32),
        ],
        mesh=_sc_mesh(),
        compiler_params=pltpu.CompilerParams(needs_layout_passes=False),
        scratch_types=[
            pltpu.VMEM((IB5, EB5), jnp.int32),
            pltpu.VMEM((IB5, EB5), jnp.int32),
            pltpu.VMEM((EB5, D), F32),
            pltpu.VMEM((EB5, D), F32),
            pltpu.VMEM((EB5, D), F32),
            pltpu.VMEM((EB5, D), F32),
            pltpu.VMEM((EB5,), F32),
            pltpu.VMEM((EB5,), F32),
            pltpu.VMEM((EB5,), F32),
            pltpu.VMEM((EB5,), F32),
            pltpu.VMEM((D,), F32),
            pltpu.SemaphoreType.DMA,
            pltpu.SemaphoreType.DMA,
            pltpu.SemaphoreType.DMA,
            pltpu.SemaphoreType.DMA,
            pltpu.VMEM_SHARED((NPAD, D), F32),
            pltpu.VMEM_SHARED((NPAD,), F32),
        ],
    )
    def gat_kernel(xl_hbm, xr_hbm, sl_hbm, att_hbm, row_hbm, col_hbm,
                   zeros2_hbm, zeros1_hbm, num_hbm, den_hbm,
                   rowb, colb, uw0, uw1, xlw0, xlw1, slw0, slw1, pv0, pv1,
                   attv, semg0, semg1, sems0, sems1, num, den):
        c = lax.axis_index("c")
        s = lax.axis_index("s")
        w = c * NS + s
        pltpu.sync_copy(zeros2_hbm, uw0)
        pltpu.sync_copy(zeros1_hbm, pv0)
        for i in range(CHUNK // EB5):
            pltpu.sync_copy(uw0, num.at[pl.ds(s * CHUNK + i * EB5, EB5)])
            pltpu.sync_copy(pv0, den.at[pl.ds(s * CHUNK + i * EB5, EB5)])
        pltpu.sync_copy(att_hbm, attv)
        plsc.subcore_barrier()

        att_chunks = [attv[pl.ds(k * 16, 16)] for k in range(D // 16)]
        iota = lax.iota(jnp.int32, 16)
        lane0 = iota == 0
        idx15 = jnp.full((16,), 15, jnp.int32)

        sets = ((uw0, xlw0, slw0, pv0, semg0, sems0),
                (uw1, xlw1, slw1, pv1, semg1, sems1))

        def issue_g(b, st):
            uwx, xlwx, slwx, _, gsem, _ = st
            pltpu.async_copy(xl_hbm.at[rowb.at[b]], xlwx, gsem)
            pltpu.async_copy(xr_hbm.at[colb.at[b]], uwx, gsem)
            pltpu.async_copy(sl_hbm.at[colb.at[b]], slwx, gsem)

        def drain_g(st):
            uwx, xlwx, slwx, _, gsem, _ = st
            pltpu.make_async_copy(xl_hbm.at[rowb.at[0]], xlwx, gsem).wait()
            pltpu.make_async_copy(xr_hbm.at[colb.at[0]], uwx, gsem).wait()
            pltpu.make_async_copy(sl_hbm.at[colb.at[0]], slwx, gsem).wait()

        def issue_s(b, st):
            uwx, _, _, pvx, _, ssem = st
            pltpu.async_copy(uwx, num.at[colb.at[b]], ssem, add=True)
            pltpu.async_copy(pvx, den.at[colb.at[b]], ssem, add=True)

        def drain_s(st):
            uwx, _, _, pvx, _, ssem = st
            pltpu.make_async_copy(uwx, num.at[colb.at[0]], ssem).wait()
            pltpu.make_async_copy(pvx, den.at[colb.at[0]], ssem).wait()

        def compute(st):
            uwx, xlwx, slwx, pvx, _, _ = st

            @plsc.parallel_loop(0, EB5, unroll=2)
            def crow(r):
                acc = jnp.zeros((16,), F32)
                a_chunks = []
                for k in range(D // 16):
                    a = xlwx[r, pl.ds(k * 16, 16)]
                    b = uwx[r, pl.ds(k * 16, 16)]
                    a_chunks.append(a)
                    u = a + b
                    lr = jnp.where(u >= 0.0, u, 0.2 * u)
                    acc = acc + lr * att_chunks[k]
                cum = plsc.cumsum(acc)
                sc = _lane_pick(cum, idx15)
                base = (r // 16) * 16
                slchunk = slwx[pl.ds(base, 16)]
                slr = _lane_pick(slchunk,
                                 jnp.full((16,), lax.rem(r, 16), jnp.int32))
                q = jnp.exp(sc - slr)
                for k in range(D // 16):
                    uwx[r, pl.ds(k * 16, 16)] = a_chunks[k] * q
                plsc.store_scatter(pvx, [jnp.full((16,), r, jnp.int32)], q,
                                   mask=lane0)

        def refill(rr, carry):
            pltpu.sync_copy(row_hbm.at[w, pl.ds(rr * IB5, IB5)], rowb)
            pltpu.sync_copy(col_hbm.at[w, pl.ds(rr * IB5, IB5)], colb)
            issue_g(0, sets[0])

            def inner(bb, carry2):
                b0 = bb * 2
                b1 = b0 + 1
                drain_g(sets[0])

                @pl.when(bb > 0)
                def _():
                    drain_s(sets[1])

                issue_g(b1, sets[1])
                compute(sets[0])
                issue_s(b0, sets[0])
                drain_g(sets[1])
                drain_s(sets[0])

                @pl.when(bb < IB5 // 2 - 1)
                def _():
                    issue_g(b0 + 2, sets[0])

                compute(sets[1])
                issue_s(b1, sets[1])
                return carry2

            lax.fori_loop(0, IB5 // 2, inner, 0)
            drain_s(sets[1])
            return carry

        lax.fori_loop(0, nb5 // IB5, refill, 0)
        plsc.subcore_barrier()
        for i in range(CHUNK // EB5):
            pltpu.sync_copy(num.at[pl.ds(s * CHUNK + i * EB5, EB5)], uw0)
            pltpu.sync_copy(uw0,
                            num_hbm.at[c, pl.ds(s * CHUNK + i * EB5, EB5)])
            pltpu.sync_copy(den.at[pl.ds(s * CHUNK + i * EB5, EB5)], pv0)
            pltpu.sync_copy(
                pv0, den_hbm.at[pl.ds(c * NPAD + s * CHUNK + i * EB5, EB5)])

    return gat_kernel


def _tc_prep(xp, W, d0, d1):
    def body(x_ref, w_ref, d0_ref, d1_ref, y_ref, dis_ref):
        deg = d0_ref[...] + d1_ref[...] + 1.0
        dis = lax.rsqrt(deg)
        xw = jnp.dot(x_ref[...], w_ref[...], preferred_element_type=F32)
        y_ref[...] = xw * dis
        dis_ref[...] = dis

    return pl.pallas_call(
        body,
        grid=(GRID,),
        in_specs=[
            pl.BlockSpec((BLK, D), lambda i: (i, 0)),
            pl.BlockSpec((D, D), lambda i: (0, 0)),
            pl.BlockSpec((BLK, 1), lambda i: (i, 0)),
            pl.BlockSpec((BLK, 1), lambda i: (i, 0)),
        ],
        out_specs=[
            pl.BlockSpec((BLK, D), lambda i: (i, 0)),
            pl.BlockSpec((BLK, 1), lambda i: (i, 0)),
        ],
        out_shape=[
            jax.ShapeDtypeStruct((NPAD, D), F32),
            jax.ShapeDtypeStruct((NPAD, 1), F32),
        ],
    )(xp, W, d0, d1)


def _tc_mid(acc0, acc1, y, dis, b_gcn, ln_g, ln_b, Wl, Wr, att):
    def body(a0, a1, y_ref, dis_ref, bg, lg, lb, wl, wr, at,
             xl_ref, xr_ref, sl_ref):
        pre = (a0[...] + a1[...] + y_ref[...]) * dis_ref[...] + bg[...]
        hr = jnp.maximum(pre, 0.0)
        mu = jnp.mean(hr, axis=1, keepdims=True)
        xc = hr - mu
        var = jnp.mean(xc * xc, axis=1, keepdims=True)
        h = xc * lax.rsqrt(var + 1e-5) * lg[...] + lb[...]
        xl = jnp.dot(h, wl[...], preferred_element_type=F32)
        xr = jnp.dot(h, wr[...], preferred_element_type=F32)
        u = xl + xr
        lr = jnp.where(u >= 0.0, u, 0.2 * u)
        sl_ref[...] = jnp.sum(lr * at[...], axis=1, keepdims=True)
        xl_ref[...] = xl
        xr_ref[...] = xr

    return pl.pallas_call(
        body,
        grid=(GRID,),
        in_specs=[
            pl.BlockSpec((BLK, D), lambda i: (i, 0)),
            pl.BlockSpec((BLK, D), lambda i: (i, 0)),
            pl.BlockSpec((BLK, D), lambda i: (i, 0)),
            pl.BlockSpec((BLK, 1), lambda i: (i, 0)),
            pl.BlockSpec((1, D), lambda i: (0, 0)),
            pl.BlockSpec((1, D), lambda i: (0, 0)),
            pl.BlockSpec((1, D), lambda i: (0, 0)),
            pl.BlockSpec((D, D), lambda i: (0, 0)),
            pl.BlockSpec((D, D), lambda i: (0, 0)),
            pl.BlockSpec((1, D), lambda i: (0, 0)),
        ],
        out_specs=[
            pl.BlockSpec((BLK, D), lambda i: (i, 0)),
            pl.BlockSpec((BLK, D), lambda i: (i, 0)),
            pl.BlockSpec((BLK, 1), lambda i: (i, 0)),
        ],
        out_shape=[
            jax.ShapeDtypeStruct((NPAD, D), F32),
            jax.ShapeDtypeStruct((NPAD, D), F32),
            jax.ShapeDtypeStruct((NPAD, 1), F32),
        ],
    )(acc0, acc1, y, dis, b_gcn, ln_g, ln_b, Wl, Wr, att)


def _tc_out(num0, num1, xl, den0, den1, b_gat):
    def body(n0, n1, xl_ref, d0_ref, d1_ref, bg, out_ref):
        dent = d0_ref[...] + d1_ref[...] + 1.0 + 1e-16
        o = (n0[...] + n1[...] + xl_ref[...]) / dent + bg[...]
        out_ref[...] = jnp.maximum(o, 0.0)

    return pl.pallas_call(
        body,
        grid=(GRID,),
        in_specs=[
            pl.BlockSpec((BLK, D), lambda i: (i, 0)),
            pl.BlockSpec((BLK, D), lambda i: (i, 0)),
            pl.BlockSpec((BLK, D), lambda i: (i, 0)),
            pl.BlockSpec((BLK, 1), lambda i: (i, 0)),
            pl.BlockSpec((BLK, 1), lambda i: (i, 0)),
            pl.BlockSpec((1, D), lambda i: (0, 0)),
        ],
        out_specs=pl.BlockSpec((BLK, D), lambda i: (i, 0)),
        out_shape=jax.ShapeDtypeStruct((NPAD, D), F32),
    )(num0, num1, xl, den0, den1, b_gat)


def kernel(x, edge_index, W_gcn, b_gcn, ln_g, ln_b, Wl, Wr, att, b_gat):
    ei = edge_index.astype(jnp.int32)
    E = ei.shape[1]
    per = NW * EB * 16
    nb = 16 * (-(-E // per))
    Ep = NW * nb * EB
    nb5 = Ep // (NW * EB5)
    pad = Ep - E
    dmy = N + (jnp.arange(pad, dtype=jnp.int32) % (NPAD - N))
    rowf = jnp.concatenate([ei[0], dmy])
    colf = jnp.concatenate([ei[1], dmy])
    row = rowf.reshape(NW, nb, EB)
    col = colf.reshape(NW, nb, EB)
    row5 = rowf.reshape(NW, nb5, EB5)
    col5 = colf.reshape(NW, nb5, EB5)
    zeros2 = jnp.zeros((EB, D), F32)
    zeros1 = jnp.zeros((EB,), F32)
    xp = jnp.zeros((NPAD, D), F32).at[:N].set(x)

    degp = _deg_call(nb)(col, zeros1)
    y, dis = _tc_prep(xp, W_gcn, degp[:NPAD][:, None], degp[NPAD:][:, None])
    accp = _gcn_agg_call(nb)(y, row, col, zeros2)
    xl, xr, sl = _tc_mid(accp[0], accp[1], y, dis,
                         b_gcn.reshape(1, D), ln_g.reshape(1, D),
                         ln_b.reshape(1, D), Wl, Wr, att.reshape(1, D))
    nump, denp = _gat_edge_call(nb5)(xl, xr, sl.reshape(NPAD), att,
                                     row5, col5, zeros2[:EB5], zeros1[:EB5])
    out = _tc_out(nump[0], nump[1], xl,
                  denp[:NPAD][:, None], denp[NPAD:][:, None],
                  b_gat.reshape(1, D))
    return out[:N]
